# baseline (speedup 1.0000x reference)
import jax
import jax.numpy as jnp
from jax.experimental import pallas as pl
from jax.experimental.pallas import tpu as pltpu


def _make_lstm_body(seq_len, b_pad, d_pad, hid, n_streams):
    bs = b_pad // n_streams

    def body(x_ref, wcat_ref, b_ref,
             hid_ref, cell_ref,
             seq_ref):
        b = b_ref[0]

        @pl.when(pl.program_id(0) == 0)
        def _():
            for t in range(seq_len):
                seq_ref[t * b_pad:(t + 1) * b_pad, :] = (
                    x_ref[:, t * d_pad:(t + 1) * d_pad].astype(jnp.bfloat16))

        h = [jnp.zeros((bs, hid), jnp.float32) for _ in range(n_streams)]
        c = [jnp.zeros((bs, hid), jnp.float32) for _ in range(n_streams)]
        hb = [None] * n_streams
        for t in range(seq_len):
            for s in range(n_streams):
                r0 = t * b_pad + s * bs
                s_t = seq_ref[r0:r0 + bs, :]
                if t == 0:
                    g = jnp.dot(s_t, wcat_ref[0, hid:, :],
                                preferred_element_type=jnp.float32) + b
                else:
                    lhs = jnp.concatenate([hb[s], s_t], axis=1)
                    g = jnp.dot(lhs, wcat_ref[0],
                                preferred_element_type=jnp.float32) + b

                ti = jnp.tanh(g[:, 0 * hid:1 * hid])
                tf = jnp.tanh(g[:, 1 * hid:2 * hid])
                tg = jnp.tanh(g[:, 2 * hid:3 * hid])
                to = jnp.tanh(g[:, 3 * hid:4 * hid])

                c[s] = 0.5 * ((tf * c[s] + c[s]) + (ti * tg + tg))
                tc = jnp.tanh(c[s])
                h[s] = 0.5 * (to * tc + tc)
                hb[s] = h[s].astype(jnp.bfloat16)
                seq_ref[r0:r0 + bs, :hid] = hb[s]

        hid_ref[0] = jnp.concatenate(h, axis=0) if n_streams > 1 else h[0]
        cell_ref[0] = jnp.concatenate(c, axis=0) if n_streams > 1 else c[0]

    return body


def kernel(x, w_ih_all, w_hh_all, b_all):
    num_layers, d_pad, four_h = w_ih_all.shape
    hid = four_h // 4
    B, T, D = x.shape

    n_streams = 2
    b_pad = max(8 * n_streams, -(-B // (8 * n_streams)) * (8 * n_streams))
    if b_pad != B or d_pad != D:
        x = jnp.pad(x, ((0, b_pad - B), (0, 0), (0, d_pad - D)))
    x2 = x.reshape(b_pad, T * d_pad)

    sc = jnp.concatenate([
        jnp.full((2 * hid,), 0.5, jnp.float32),
        jnp.ones((hid,), jnp.float32),
        jnp.full((hid,), 0.5, jnp.float32),
    ])
    wcat_all = (jnp.concatenate([w_hh_all, w_ih_all], axis=1) * sc
                ).astype(jnp.bfloat16)
    b_sc = b_all * sc

    body = _make_lstm_body(T, b_pad, d_pad, hid, n_streams)

    hidden, cell = pl.pallas_call(
        body,
        grid=(num_layers,),
        in_specs=[
            pl.BlockSpec((b_pad, T * d_pad), lambda l: (0, 0)),
            pl.BlockSpec((1, hid + d_pad, four_h), lambda l: (l, 0, 0)),
            pl.BlockSpec((1, 1, four_h), lambda l: (l, 0, 0)),
        ],
        out_specs=[
            pl.BlockSpec((1, b_pad, hid), lambda l: (l, 0, 0)),
            pl.BlockSpec((1, b_pad, hid), lambda l: (l, 0, 0)),
        ],
        out_shape=(
            jax.ShapeDtypeStruct((num_layers, b_pad, hid), jnp.float32),
            jax.ShapeDtypeStruct((num_layers, b_pad, hid), jnp.float32),
        ),
        scratch_shapes=[
            pltpu.VMEM((T * b_pad, d_pad), jnp.bfloat16),
        ],
        compiler_params=pltpu.CompilerParams(
            dimension_semantics=("arbitrary",)),
    )(x2, wcat_all, b_sc)

    if b_pad != B:
        hidden, cell = hidden[:, :B, :], cell[:, :B, :]
    return hidden, cell
